# Initial kernel scaffold; baseline (speedup 1.0000x reference)
import jax
import jax.numpy as jnp
from jax.experimental import pallas as pl


def kernel(x, edge_index, W1_l, b1_l, W1_r, W2_l, b2_l, W2_r):
    raise NotImplementedError("write your pallas kernel here")


# speedup vs baseline: 15.0293x; 15.0293x over previous
import functools

import jax
import jax.numpy as jnp
from jax import lax
from jax.experimental import pallas as pl
from jax.experimental.pallas import tpu as pltpu
from jax.experimental.pallas import tpu_sc as plsc

_N = 100000
_NPAD = 102400
_D = 16
_LANE = 128
_GRP = 8
_NGRP = 49
_NC, _NS = 2, 16
_NTILE = _NC * _NS
_TILE_ROWS = _GRP * _NGRP
_EROWS = _TILE_ROWS * _NTILE
_EPAD = _EROWS * _LANE
_TROWS = _NPAD // _NS
_ZROWS = 640


_sc_mesh = plsc.VectorSubcoreMesh(core_axis_name="c", subcore_axis_name="s")
_sc_params = pltpu.CompilerParams(use_tc_tiling_on_sc=False)


def _make_edge_pass():
    scratch = [
        pltpu.VMEM_SHARED((_NPAD, _D), jnp.float32),
        pltpu.VMEM((_GRP, _LANE), jnp.int32),
        pltpu.VMEM((_GRP, _LANE), jnp.int32),
        pltpu.VMEM((_GRP, _LANE, _D), jnp.float32),
        pltpu.VMEM((_ZROWS, _D), jnp.float32),
        pltpu.SemaphoreType.DMA,
    ]

    def body(table, srcr, dstr, acc_out, acc_sh, src_v, dst_v, rows_v,
             zacc_v, sem):
        c = lax.axis_index("c")
        s = lax.axis_index("s")
        t0 = s * _TROWS

        def zrow(i, carry):
            zacc_v[i, :] = jnp.zeros((_D,), jnp.float32)
            return carry
        lax.fori_loop(0, _ZROWS, zrow, 0)

        def zcopy(g, carry):
            pltpu.sync_copy(zacc_v, acc_sh.at[pl.ds(t0 + g * _ZROWS, _ZROWS)])
            return carry
        lax.fori_loop(0, _TROWS // _ZROWS, zcopy, 0)

        plsc.subcore_barrier()

        base = (c * _NS + s) * _TILE_ROWS

        def group(g, carry):
            r0 = base + g * _GRP
            pltpu.sync_copy(srcr.at[pl.ds(r0, _GRP)], src_v)
            pltpu.sync_copy(dstr.at[pl.ds(r0, _GRP)], dst_v)
            descs = [
                pltpu.async_copy(table.at[src_v.at[j]], rows_v.at[j], sem)
                for j in range(_GRP)
            ]
            for d in descs:
                d.wait()
            for j in range(_GRP):
                pltpu.sync_copy(rows_v.at[j], acc_sh.at[dst_v.at[j]], add=True)
            return carry
        lax.fori_loop(0, _NGRP, group, 0)

        plsc.subcore_barrier()
        pltpu.sync_copy(acc_sh.at[pl.ds(t0, _TROWS)],
                        acc_out.at[c, pl.ds(t0, _TROWS)])

    return pl.kernel(
        body, out_type=jax.ShapeDtypeStruct((_NC, _NPAD, _D), jnp.float32),
        mesh=_sc_mesh, scratch_types=scratch, compiler_params=_sc_params)


def _make_deg_pass():
    scratch = [
        pltpu.VMEM_SHARED((_NPAD,), jnp.float32),
        pltpu.VMEM((_GRP, _LANE), jnp.int32),
        pltpu.VMEM((_TROWS,), jnp.float32),
        pltpu.VMEM((_LANE,), jnp.float32),
    ]

    def body(dstr, deg_out, deg_sh, dst_v, zdeg_v, ones_v):
        c = lax.axis_index("c")
        s = lax.axis_index("s")
        t0 = s * _TROWS

        def zdrow(i, carry):
            zdeg_v[pl.ds(i * 16, 16)] = jnp.zeros((16,), jnp.float32)
            return carry
        lax.fori_loop(0, _TROWS // 16, zdrow, 0)
        pltpu.sync_copy(zdeg_v, deg_sh.at[pl.ds(t0, _TROWS)])

        def orow(i, carry):
            ones_v[pl.ds(i * 16, 16)] = jnp.ones((16,), jnp.float32)
            return carry
        lax.fori_loop(0, _LANE // 16, orow, 0)

        plsc.subcore_barrier()

        base = (c * _NS + s) * _TILE_ROWS

        def group(g, carry):
            r0 = base + g * _GRP
            pltpu.sync_copy(dstr.at[pl.ds(r0, _GRP)], dst_v)
            for j in range(_GRP):
                pltpu.sync_copy(ones_v, deg_sh.at[dst_v.at[j]], add=True)
            return carry
        lax.fori_loop(0, _NGRP, group, 0)

        plsc.subcore_barrier()
        pltpu.sync_copy(deg_sh.at[pl.ds(t0, _TROWS)],
                        deg_out.at[c, pl.ds(t0, _TROWS)])

    return pl.kernel(
        body, out_type=jax.ShapeDtypeStruct((_NC, _NPAD), jnp.float32),
        mesh=_sc_mesh, scratch_types=scratch, compiler_params=_sc_params)


_edge_pass = _make_edge_pass()
_deg_pass = _make_deg_pass()

_BLK = 1024
_NBLK = _NPAD // _BLK


def _tc1_body(p_ref, d_ref, x_ref, w1l, b1l, w1r, w2l, b2l, w2r,
              g_ref, r_ref):
    p = p_ref[0] + p_ref[1]
    d = d_ref[0] + d_ref[1]
    dinv = 1.0 / jnp.maximum(d, 1.0)
    mean = p * dinv[:, None]
    h = jnp.dot(mean, w1l[...], preferred_element_type=jnp.float32)
    h = h + b1l[...] + jnp.dot(x_ref[...], w1r[...],
                               preferred_element_type=jnp.float32)
    h = jnp.maximum(h, 0.0)
    g_ref[...] = jnp.dot(h, w2l[...], preferred_element_type=jnp.float32)
    r_ref[...] = jnp.dot(h, w2r[...], preferred_element_type=jnp.float32) \
        + b2l[...]


def _tc2_body(q_ref, d_ref, r_ref, o_ref):
    q = q_ref[0] + q_ref[1]
    d = d_ref[0] + d_ref[1]
    o_ref[...] = q * (1.0 / jnp.maximum(d, 1.0))[:, None] + r_ref[...]


_row_spec = pl.BlockSpec((_BLK, _D), lambda i: (i, 0))
_part_spec = pl.BlockSpec((_NC, _BLK, _D), lambda i: (0, i, 0))
_deg_spec = pl.BlockSpec((_NC, _BLK), lambda i: (0, i))


def _w_spec(r, c):
    return pl.BlockSpec((r, c), lambda i: (0, 0))


_tc1 = pl.pallas_call(
    _tc1_body,
    grid=(_NBLK,),
    in_specs=[
        _part_spec, _deg_spec, _row_spec,
        _w_spec(16, 32), _w_spec(1, 32), _w_spec(16, 32),
        _w_spec(32, 16), _w_spec(1, 16), _w_spec(32, 16),
    ],
    out_specs=[_row_spec, _row_spec],
    out_shape=[jax.ShapeDtypeStruct((_NPAD, _D), jnp.float32)] * 2,
)

_tc2 = pl.pallas_call(
    _tc2_body,
    grid=(_NBLK,),
    in_specs=[_part_spec, _deg_spec, _row_spec],
    out_specs=_row_spec,
    out_shape=jax.ShapeDtypeStruct((_NPAD, _D), jnp.float32),
)


def kernel(x, edge_index, W1_l, b1_l, W1_r, W2_l, b2_l, W2_r):
    src = edge_index[0]
    dst = edge_index[1]
    pad = _EPAD - src.shape[0]
    src_p = jnp.concatenate(
        [src, jnp.zeros((pad,), jnp.int32)]).reshape(_EROWS, _LANE)
    dst_p = jnp.concatenate(
        [dst, jnp.full((pad,), _N, jnp.int32)]).reshape(_EROWS, _LANE)
    x_pad = jnp.pad(x, ((0, _NPAD - _N), (0, 0)))

    p1 = _edge_pass(x_pad, src_p, dst_p)
    deg = _deg_pass(dst_p)
    g, r0 = _tc1(p1, deg, x_pad, W1_l, b1_l.reshape(1, -1), W1_r,
                 W2_l, b2_l.reshape(1, -1), W2_r)
    p2 = _edge_pass(g, src_p, dst_p)
    out = _tc2(p2, deg, r0)
    return out[:_N]


# speedup vs baseline: 20.4429x; 1.3602x over previous
import functools

import jax
import jax.numpy as jnp
from jax import lax
from jax.experimental import pallas as pl
from jax.experimental.pallas import tpu as pltpu
from jax.experimental.pallas import tpu_sc as plsc

_N = 100000
_NPAD = 102400
_D = 16
_LANE = 128
_GRP = 8
_NGRP = 49
_NC, _NS = 2, 16
_NTILE = _NC * _NS
_TILE_ROWS = _GRP * _NGRP
_EROWS = _TILE_ROWS * _NTILE
_EPAD = _EROWS * _LANE
_TROWS = _NPAD // _NS
_ZROWS = 640


_sc_mesh = plsc.VectorSubcoreMesh(core_axis_name="c", subcore_axis_name="s")
_sc_params = pltpu.CompilerParams(use_tc_tiling_on_sc=False)


def _make_edge_pass():
    scratch = [
        pltpu.VMEM_SHARED((_NPAD, _D), jnp.float32),
        pltpu.VMEM((_GRP, _LANE), jnp.int32),
        pltpu.VMEM((_GRP, _LANE), jnp.int32),
        pltpu.VMEM((_GRP, _LANE, _D), jnp.float32),
        pltpu.VMEM((_ZROWS, _D), jnp.float32),
        pltpu.SemaphoreType.DMA,
    ]

    def body(table, srcr, dstr, acc_out, acc_sh, src_v, dst_v, rows_v,
             zacc_v, sem):
        c = lax.axis_index("c")
        s = lax.axis_index("s")
        t0 = s * _TROWS

        def zrow(i, carry):
            zacc_v[i, :] = jnp.zeros((_D,), jnp.float32)
            return carry
        lax.fori_loop(0, _ZROWS, zrow, 0)

        def zcopy(g, carry):
            pltpu.sync_copy(zacc_v, acc_sh.at[pl.ds(t0 + g * _ZROWS, _ZROWS)])
            return carry
        lax.fori_loop(0, _TROWS // _ZROWS, zcopy, 0)

        plsc.subcore_barrier()

        base = (c * _NS + s) * _TILE_ROWS

        def group(g, carry):
            r0 = base + g * _GRP
            pltpu.sync_copy(srcr.at[pl.ds(r0, _GRP)], src_v)
            pltpu.sync_copy(dstr.at[pl.ds(r0, _GRP)], dst_v)
            descs = [
                pltpu.async_copy(table.at[src_v.at[j]], rows_v.at[j], sem)
                for j in range(_GRP)
            ]
            for d in descs:
                d.wait()
            for j in range(_GRP):
                pltpu.sync_copy(rows_v.at[j], acc_sh.at[dst_v.at[j]], add=True)
            return carry
        lax.fori_loop(0, _NGRP, group, 0)

        plsc.subcore_barrier()
        pltpu.sync_copy(acc_sh.at[pl.ds(t0, _TROWS)],
                        acc_out.at[c, pl.ds(t0, _TROWS)])

    return pl.kernel(
        body, out_type=jax.ShapeDtypeStruct((_NC, _NPAD, _D), jnp.float32),
        mesh=_sc_mesh, scratch_types=scratch, compiler_params=_sc_params)


def _make_deg_pass():
    scratch = [
        pltpu.VMEM_SHARED((_NPAD,), jnp.float32),
        pltpu.VMEM((_GRP, _LANE), jnp.int32),
        pltpu.VMEM((_TROWS,), jnp.float32),
        pltpu.VMEM((_LANE,), jnp.float32),
    ]

    def body(dstr, deg_out, deg_sh, dst_v, zdeg_v, ones_v):
        c = lax.axis_index("c")
        s = lax.axis_index("s")
        t0 = s * _TROWS

        def zdrow(i, carry):
            zdeg_v[pl.ds(i * 16, 16)] = jnp.zeros((16,), jnp.float32)
            return carry
        lax.fori_loop(0, _TROWS // 16, zdrow, 0)
        pltpu.sync_copy(zdeg_v, deg_sh.at[pl.ds(t0, _TROWS)])

        def orow(i, carry):
            ones_v[pl.ds(i * 16, 16)] = jnp.ones((16,), jnp.float32)
            return carry
        lax.fori_loop(0, _LANE // 16, orow, 0)

        plsc.subcore_barrier()

        base = (c * _NS + s) * _TILE_ROWS

        def group(g, carry):
            r0 = base + g * _GRP
            pltpu.sync_copy(dstr.at[pl.ds(r0, _GRP)], dst_v)
            for j in range(_GRP):
                pltpu.sync_copy(ones_v, deg_sh.at[dst_v.at[j]], add=True)
            return carry
        lax.fori_loop(0, _NGRP, group, 0)

        plsc.subcore_barrier()
        pltpu.sync_copy(deg_sh.at[pl.ds(t0, _TROWS)],
                        deg_out.at[c, pl.ds(t0, _TROWS)])

    return pl.kernel(
        body, out_type=jax.ShapeDtypeStruct((_NC, _NPAD), jnp.float32),
        mesh=_sc_mesh, scratch_types=scratch, compiler_params=_sc_params)


def _make_dinv_expand():
    npt = _NPAD // _NTILE
    scratch = [
        pltpu.VMEM((npt,), jnp.float32),
        pltpu.VMEM((npt,), jnp.float32),
        pltpu.VMEM((npt * 16,), jnp.float32),
    ]

    def body(degp, out, d0_v, d1_v, out_v):
        c = lax.axis_index("c")
        s = lax.axis_index("s")
        base = (c * _NS + s) * npt
        pltpu.sync_copy(degp.at[0, pl.ds(base, npt)], d0_v)
        pltpu.sync_copy(degp.at[1, pl.ds(base, npt)], d1_v)

        dnums = lax.GatherDimensionNumbers(
            offset_dims=(), collapsed_slice_dims=(0,), start_index_map=(0,))

        def vr(i, carry):
            v = d0_v[pl.ds(i * 16, 16)] + d1_v[pl.ds(i * 16, 16)]
            dinv = 1.0 / jnp.maximum(v, 1.0)
            for lane in range(16):
                idx = jnp.full((16, 1), lane, jnp.int32)
                out_v[pl.ds((i * 16 + lane) * 16, 16)] = lax.gather(
                    dinv, idx, dnums, slice_sizes=(1,),
                    mode=lax.GatherScatterMode.PROMISE_IN_BOUNDS)
            return carry
        lax.fori_loop(0, npt // 16, vr, 0)
        pltpu.sync_copy(out_v, out.at[pl.ds(base * 16, npt * 16)])

    return pl.kernel(
        body, out_type=jax.ShapeDtypeStruct((_NPAD * 16,), jnp.float32),
        mesh=_sc_mesh, scratch_types=scratch, compiler_params=_sc_params)


_edge_pass = _make_edge_pass()
_deg_pass = _make_deg_pass()
_dinv_expand = _make_dinv_expand()

_PROWS = _NPAD * _D // 128
_PBLK = 256
_PGRID = _PROWS // _PBLK


def _tc1_body(p_ref, dinv_ref, x_ref, k1l, b1, k1r, k2l, b2, k2r,
              g_ref, r_ref):
    mean = (p_ref[0] + p_ref[1]) * dinv_ref[...]
    h = jnp.dot(mean, k1l[...], preferred_element_type=jnp.float32)
    h = h + b1[...] + jnp.dot(x_ref[...], k1r[...],
                              preferred_element_type=jnp.float32)
    h = jnp.maximum(h, 0.0)
    g_ref[...] = jnp.dot(h, k2l[...], preferred_element_type=jnp.float32)
    r_ref[...] = jnp.dot(h, k2r[...], preferred_element_type=jnp.float32) \
        + b2[...]


def _tc2_body(q_ref, dinv_ref, r_ref, o_ref):
    o_ref[...] = (q_ref[0] + q_ref[1]) * dinv_ref[...] + r_ref[...]


_prow_spec = pl.BlockSpec((_PBLK, 128), lambda i: (i, 0))
_ppart_spec = pl.BlockSpec((_NC, _PBLK, 128), lambda i: (0, i, 0))


def _w_spec(r, c):
    return pl.BlockSpec((r, c), lambda i: (0, 0))


_tc1 = pl.pallas_call(
    _tc1_body,
    grid=(_PGRID,),
    in_specs=[
        _ppart_spec, _prow_spec, _prow_spec,
        _w_spec(128, 256), _w_spec(1, 256), _w_spec(128, 256),
        _w_spec(256, 128), _w_spec(1, 128), _w_spec(256, 128),
    ],
    out_specs=[_prow_spec, _prow_spec],
    out_shape=[jax.ShapeDtypeStruct((_PROWS, 128), jnp.float32)] * 2,
)

_tc2 = pl.pallas_call(
    _tc2_body,
    grid=(_PGRID,),
    in_specs=[_ppart_spec, _prow_spec, _prow_spec],
    out_specs=_prow_spec,
    out_shape=jax.ShapeDtypeStruct((_PROWS, 128), jnp.float32),
)


def kernel(x, edge_index, W1_l, b1_l, W1_r, W2_l, b2_l, W2_r):
    src = edge_index[0]
    dst = edge_index[1]
    pad = _EPAD - src.shape[0]
    src_p = jnp.concatenate(
        [src, jnp.zeros((pad,), jnp.int32)]).reshape(_EROWS, _LANE)
    dst_p = jnp.concatenate(
        [dst, jnp.full((pad,), _N, jnp.int32)]).reshape(_EROWS, _LANE)
    x_pad = jnp.pad(x, ((0, _NPAD - _N), (0, 0)))
    eye8 = jnp.eye(8, dtype=jnp.float32)
    k1l = jnp.kron(eye8, W1_l)
    k1r = jnp.kron(eye8, W1_r)
    k2l = jnp.kron(eye8, W2_l)
    k2r = jnp.kron(eye8, W2_r)
    b1t = jnp.tile(b1_l, 8).reshape(1, -1)
    b2t = jnp.tile(b2_l, 8).reshape(1, -1)

    p1 = _edge_pass(x_pad, src_p, dst_p)
    degp = _deg_pass(dst_p)
    dinv = _dinv_expand(degp).reshape(_PROWS, 128)
    g, r0 = _tc1(p1.reshape(_NC, _PROWS, 128), dinv,
                 x_pad.reshape(_PROWS, 128),
                 k1l, b1t, k1r, k2l, b2t, k2r)
    p2 = _edge_pass(g.reshape(_NPAD, _D), src_p, dst_p)
    out = _tc2(p2.reshape(_NC, _PROWS, 128), dinv, r0)
    return out.reshape(_NPAD, _D)[:_N]


# speedup vs baseline: 22.2206x; 1.0870x over previous
import functools

import jax
import jax.numpy as jnp
from jax import lax
from jax.experimental import pallas as pl
from jax.experimental.pallas import tpu as pltpu
from jax.experimental.pallas import tpu_sc as plsc

_N = 100000
_NPAD = 100352
_D = 16
_LANE = 128
_GRP = 8
_NC, _NS = 2, 16
_NTILE = _NC * _NS
_EROWS = 1600000 // _LANE
_NFULL = _EROWS // _GRP
_TAIL_ROW = _NFULL * _GRP
_TAIL = _EROWS - _TAIL_ROW
_TROWS = _NPAD // _NS
_ZROWS = 98

_sc_mesh = plsc.VectorSubcoreMesh(core_axis_name="c", subcore_axis_name="s")
_sc_params = pltpu.CompilerParams(use_tc_tiling_on_sc=False)


def _make_edge_pass(with_deg):
    outs = [jax.ShapeDtypeStruct((_NC, _NPAD, _D), jnp.float32)]
    scratch = [
        pltpu.VMEM_SHARED((_NPAD, _D), jnp.float32),
        pltpu.VMEM((_GRP, _LANE), jnp.int32),
        pltpu.VMEM((_GRP, _LANE), jnp.int32),
        pltpu.VMEM((_TAIL, _LANE), jnp.int32),
        pltpu.VMEM((_TAIL, _LANE), jnp.int32),
        pltpu.VMEM((_GRP, _LANE, _D), jnp.float32),
        pltpu.VMEM((_ZROWS, _D), jnp.float32),
        pltpu.SemaphoreType.DMA,
    ]
    if with_deg:
        outs.append(jax.ShapeDtypeStruct((_NC, _NPAD), jnp.float32))
        scratch += [
            pltpu.VMEM_SHARED((_NPAD,), jnp.float32),
            pltpu.VMEM((_TROWS // 8,), jnp.float32),
            pltpu.VMEM((_LANE,), jnp.float32),
        ]

    def body(table, srcr, dstr, *rest):
        if with_deg:
            (acc_out, deg_out, acc_sh, src_v, dst_v, src4_v, dst4_v, rows_v,
             zacc_v, sem, deg_sh, zdeg_v, ones_v) = rest
        else:
            (acc_out, acc_sh, src_v, dst_v, src4_v, dst4_v, rows_v,
             zacc_v, sem) = rest
        c = lax.axis_index("c")
        s = lax.axis_index("s")
        w = c * _NS + s
        t0 = s * _TROWS

        def zrow(i, carry):
            zacc_v[i, :] = jnp.zeros((_D,), jnp.float32)
            return carry
        lax.fori_loop(0, _ZROWS, zrow, 0)

        def zcopy(g, carry):
            pltpu.sync_copy(zacc_v, acc_sh.at[pl.ds(t0 + g * _ZROWS, _ZROWS)])
            return carry
        lax.fori_loop(0, _TROWS // _ZROWS, zcopy, 0)

        if with_deg:
            def zdrow(i, carry):
                zdeg_v[pl.ds(i * 16, 16)] = jnp.zeros((16,), jnp.float32)
                return carry
            lax.fori_loop(0, _TROWS // 8 // 16, zdrow, 0)

            def zdcopy(g, carry):
                pltpu.sync_copy(
                    zdeg_v,
                    deg_sh.at[pl.ds(t0 + g * (_TROWS // 8), _TROWS // 8)])
                return carry
            lax.fori_loop(0, 8, zdcopy, 0)

            def orow(i, carry):
                ones_v[pl.ds(i * 16, 16)] = jnp.ones((16,), jnp.float32)
                return carry
            lax.fori_loop(0, _LANE // 16, orow, 0)

        plsc.subcore_barrier()

        ngroups = jnp.where(w < _NFULL % _NTILE, _NFULL // _NTILE + 1,
                            _NFULL // _NTILE)

        def group(g, carry):
            r0 = (g * _NTILE + w) * _GRP
            pltpu.sync_copy(srcr.at[pl.ds(r0, _GRP)], src_v)
            pltpu.sync_copy(dstr.at[pl.ds(r0, _GRP)], dst_v)
            descs = [
                pltpu.async_copy(table.at[src_v.at[j]], rows_v.at[j], sem)
                for j in range(_GRP)
            ]
            for d in descs:
                d.wait()
            for j in range(_GRP):
                pltpu.sync_copy(rows_v.at[j], acc_sh.at[dst_v.at[j]],
                                add=True)
            if with_deg:
                for j in range(_GRP):
                    pltpu.sync_copy(ones_v, deg_sh.at[dst_v.at[j]], add=True)
            return carry
        lax.fori_loop(0, ngroups, group, 0)

        @pl.when(w == _NTILE - 1)
        def _tail():
            pltpu.sync_copy(srcr.at[pl.ds(_TAIL_ROW, _TAIL)], src4_v)
            pltpu.sync_copy(dstr.at[pl.ds(_TAIL_ROW, _TAIL)], dst4_v)
            descs = [
                pltpu.async_copy(table.at[src4_v.at[j]], rows_v.at[j], sem)
                for j in range(_TAIL)
            ]
            for d in descs:
                d.wait()
            for j in range(_TAIL):
                pltpu.sync_copy(rows_v.at[j], acc_sh.at[dst4_v.at[j]],
                                add=True)
            if with_deg:
                for j in range(_TAIL):
                    pltpu.sync_copy(ones_v, deg_sh.at[dst4_v.at[j]],
                                    add=True)

        plsc.subcore_barrier()
        pltpu.sync_copy(acc_sh.at[pl.ds(t0, _TROWS)],
                        acc_out.at[c, pl.ds(t0, _TROWS)])
        if with_deg:
            pltpu.sync_copy(deg_sh.at[pl.ds(t0, _TROWS)],
                            deg_out.at[c, pl.ds(t0, _TROWS)])

    return pl.kernel(
        body,
        out_type=tuple(outs) if with_deg else outs[0],
        mesh=_sc_mesh, scratch_types=scratch, compiler_params=_sc_params)


def _make_dinv_expand():
    npt = _NPAD // _NTILE
    scratch = [
        pltpu.VMEM((npt,), jnp.float32),
        pltpu.VMEM((npt,), jnp.float32),
        pltpu.VMEM((npt * 16,), jnp.float32),
    ]

    def body(degp, out, d0_v, d1_v, out_v):
        c = lax.axis_index("c")
        s = lax.axis_index("s")
        base = (c * _NS + s) * npt
        pltpu.sync_copy(degp.at[0, pl.ds(base, npt)], d0_v)
        pltpu.sync_copy(degp.at[1, pl.ds(base, npt)], d1_v)

        dnums = lax.GatherDimensionNumbers(
            offset_dims=(), collapsed_slice_dims=(0,), start_index_map=(0,))

        def vr(i, carry):
            v = d0_v[pl.ds(i * 16, 16)] + d1_v[pl.ds(i * 16, 16)]
            dinv = 1.0 / jnp.maximum(v, 1.0)
            for lane in range(16):
                idx = jnp.full((16, 1), lane, jnp.int32)
                out_v[pl.ds((i * 16 + lane) * 16, 16)] = lax.gather(
                    dinv, idx, dnums, slice_sizes=(1,),
                    mode=lax.GatherScatterMode.PROMISE_IN_BOUNDS)
            return carry
        lax.fori_loop(0, npt // 16, vr, 0)
        pltpu.sync_copy(out_v, out.at[pl.ds(base * 16, npt * 16)])

    return pl.kernel(
        body, out_type=jax.ShapeDtypeStruct((_NPAD * 16,), jnp.float32),
        mesh=_sc_mesh, scratch_types=scratch, compiler_params=_sc_params)


_edge_pass_deg = _make_edge_pass(True)
_edge_pass = _make_edge_pass(False)
_dinv_expand = _make_dinv_expand()

_PROWS = _NPAD * _D // 128
_XROWS = _N * _D // 128
_PBLK = 256
_PGRID = _PROWS // _PBLK


def _tc1_body(p_ref, dinv_ref, x_ref, k1, b1, k2, b2, g_ref, r_ref):
    mean = (p_ref[0] + p_ref[1]) * dinv_ref[...]
    a = jnp.concatenate([mean, x_ref[...]], axis=1)
    h = jnp.dot(a, k1[...], preferred_element_type=jnp.float32) + b1[...]
    h = jnp.maximum(h, 0.0)
    gr = jnp.dot(h, k2[...], preferred_element_type=jnp.float32)
    g_ref[...] = gr[:, :128]
    r_ref[...] = gr[:, 128:] + b2[...]


def _tc2_body(q_ref, dinv_ref, r_ref, o_ref):
    o_ref[...] = (q_ref[0] + q_ref[1]) * dinv_ref[...] + r_ref[...]


_prow_spec = pl.BlockSpec((_PBLK, 128), lambda i: (i, 0))
_ppart_spec = pl.BlockSpec((_NC, _PBLK, 128), lambda i: (0, i, 0))


def _w_spec(r, c):
    return pl.BlockSpec((r, c), lambda i: (0, 0))


_tc1 = pl.pallas_call(
    _tc1_body,
    grid=(_PGRID,),
    in_specs=[
        _ppart_spec, _prow_spec, _prow_spec,
        _w_spec(256, 256), _w_spec(1, 256),
        _w_spec(256, 256), _w_spec(1, 128),
    ],
    out_specs=[_prow_spec, _prow_spec],
    out_shape=[jax.ShapeDtypeStruct((_PROWS, 128), jnp.float32)] * 2,
)

_tc2 = pl.pallas_call(
    _tc2_body,
    grid=(_PGRID,),
    in_specs=[_ppart_spec, _prow_spec, _prow_spec],
    out_specs=pl.BlockSpec((_PBLK, 128), lambda i: (i, 0)),
    out_shape=jax.ShapeDtypeStruct((_XROWS, 128), jnp.float32),
)


def kernel(x, edge_index, W1_l, b1_l, W1_r, W2_l, b2_l, W2_r):
    src_p = edge_index[0].reshape(_EROWS, _LANE)
    dst_p = edge_index[1].reshape(_EROWS, _LANE)
    eye8 = jnp.eye(8, dtype=jnp.float32)
    k1 = jnp.concatenate(
        [jnp.kron(eye8, W1_l), jnp.kron(eye8, W1_r)], axis=0)
    k2 = jnp.concatenate(
        [jnp.kron(eye8, W2_l), jnp.kron(eye8, W2_r)], axis=1)
    b1t = jnp.tile(b1_l, 8).reshape(1, -1)
    b2t = jnp.tile(b2_l, 8).reshape(1, -1)

    p1, degp = _edge_pass_deg(x, src_p, dst_p)
    dinv = _dinv_expand(degp).reshape(_PROWS, 128)
    g, r0 = _tc1(p1.reshape(_NC, _PROWS, 128), dinv,
                 x.reshape(_XROWS, 128), k1, b1t, k2, b2t)
    p2 = _edge_pass(g.reshape(_NPAD, _D), src_p, dst_p)
    out = _tc2(p2.reshape(_NC, _PROWS, 128), dinv, r0)
    return out.reshape(_N, _D)


# speedup vs baseline: 27.9446x; 1.2576x over previous
import functools

import jax
import jax.numpy as jnp
from jax import lax
from jax.experimental import pallas as pl
from jax.experimental.pallas import tpu as pltpu
from jax.experimental.pallas import tpu_sc as plsc

_N = 100000
_NPAD = 100352
_D = 16
_LANE = 128
_GRP = 8
_NC, _NS = 2, 16
_NTILE = _NC * _NS
_EROWS = 1600000 // _LANE
_NFULL = _EROWS // _GRP
_TAIL_ROW = _NFULL * _GRP
_TAIL = _EROWS - _TAIL_ROW
_TROWS = _NPAD // _NS
_ZROWS = 98

_sc_mesh = plsc.VectorSubcoreMesh(core_axis_name="c", subcore_axis_name="s")
_sc_params = pltpu.CompilerParams(use_tc_tiling_on_sc=False)


def _make_edge_pass(with_deg):
    outs = [jax.ShapeDtypeStruct((_NC, _NPAD, _D), jnp.float32)]
    scratch = [
        pltpu.VMEM_SHARED((_NPAD, _D), jnp.float32),
        pltpu.VMEM((_GRP, _LANE), jnp.int32),
        pltpu.VMEM((_GRP, _LANE), jnp.int32),
        pltpu.VMEM((_TAIL, _LANE), jnp.int32),
        pltpu.VMEM((_TAIL, _LANE), jnp.int32),
        pltpu.VMEM((_GRP, _LANE, _D), jnp.float32),
        pltpu.VMEM((_ZROWS, _D), jnp.float32),
        pltpu.SemaphoreType.DMA,
        pltpu.SemaphoreType.DMA,
        pltpu.SemaphoreType.DMA,
    ]
    if with_deg:
        outs.append(jax.ShapeDtypeStruct((_NC, _NPAD), jnp.float32))
        scratch += [
            pltpu.VMEM_SHARED((_NPAD,), jnp.float32),
            pltpu.VMEM((_TROWS // 8,), jnp.float32),
            pltpu.VMEM((_LANE,), jnp.float32),
        ]

    def body(edges, table, *rest):
        if with_deg:
            (acc_out, deg_out, acc_sh, src_v, dst_v, src4_v, dst4_v, rows_v,
             zacc_v, sem_a, sem_b, sem_c, deg_sh, zdeg_v, ones_v) = rest
        else:
            (acc_out, acc_sh, src_v, dst_v, src4_v, dst4_v, rows_v,
             zacc_v, sem_a, sem_b, sem_c) = rest
        c = lax.axis_index("c")
        s = lax.axis_index("s")
        w = c * _NS + s
        t0 = s * _TROWS

        def zrow(i, carry):
            zacc_v[i, :] = jnp.zeros((_D,), jnp.float32)
            return carry
        lax.fori_loop(0, _ZROWS, zrow, 0)

        def zcopy(g, carry):
            pltpu.sync_copy(zacc_v, acc_sh.at[pl.ds(t0 + g * _ZROWS, _ZROWS)])
            return carry
        lax.fori_loop(0, _TROWS // _ZROWS, zcopy, 0)

        if with_deg:
            def zdrow(i, carry):
                zdeg_v[pl.ds(i * 16, 16)] = jnp.zeros((16,), jnp.float32)
                return carry
            lax.fori_loop(0, _TROWS // 8 // 16, zdrow, 0)

            def zdcopy(g, carry):
                pltpu.sync_copy(
                    zdeg_v,
                    deg_sh.at[pl.ds(t0 + g * (_TROWS // 8), _TROWS // 8)])
                return carry
            lax.fori_loop(0, 8, zdcopy, 0)

            def orow(i, carry):
                ones_v[pl.ds(i * 16, 16)] = jnp.ones((16,), jnp.float32)
                return carry
            lax.fori_loop(0, _LANE // 16, orow, 0)

        plsc.subcore_barrier()

        ngroups = jnp.where(w < _NFULL % _NTILE, _NFULL // _NTILE + 1,
                            _NFULL // _NTILE)

        half = _GRP // 2

        def group(g, carry):
            r0 = (g * _NTILE + w) * _GRP
            pltpu.sync_copy(edges.at[0, pl.ds(r0, _GRP)], src_v)
            pltpu.sync_copy(edges.at[1, pl.ds(r0, _GRP)], dst_v)
            ga = [
                pltpu.async_copy(table.at[src_v.at[j]], rows_v.at[j], sem_a)
                for j in range(half)
            ]
            gb = [
                pltpu.async_copy(table.at[src_v.at[j]], rows_v.at[j], sem_b)
                for j in range(half, _GRP)
            ]
            sc = []
            for d in ga:
                d.wait()
            for j in range(half):
                sc.append(pltpu.async_copy(
                    rows_v.at[j], acc_sh.at[dst_v.at[j]], sem_c, add=True))
                if with_deg:
                    sc.append(pltpu.async_copy(
                        ones_v, deg_sh.at[dst_v.at[j]], sem_c, add=True))
            for d in gb:
                d.wait()
            for j in range(half, _GRP):
                sc.append(pltpu.async_copy(
                    rows_v.at[j], acc_sh.at[dst_v.at[j]], sem_c, add=True))
                if with_deg:
                    sc.append(pltpu.async_copy(
                        ones_v, deg_sh.at[dst_v.at[j]], sem_c, add=True))
            for d in sc:
                d.wait()
            return carry
        lax.fori_loop(0, ngroups, group, 0)

        @pl.when(w == _NTILE - 1)
        def _tail():
            pltpu.sync_copy(edges.at[0, pl.ds(_TAIL_ROW, _TAIL)], src4_v)
            pltpu.sync_copy(edges.at[1, pl.ds(_TAIL_ROW, _TAIL)], dst4_v)
            descs = [
                pltpu.async_copy(table.at[src4_v.at[j]], rows_v.at[j], sem_a)
                for j in range(_TAIL)
            ]
            for d in descs:
                d.wait()
            for j in range(_TAIL):
                pltpu.sync_copy(rows_v.at[j], acc_sh.at[dst4_v.at[j]],
                                add=True)
            if with_deg:
                for j in range(_TAIL):
                    pltpu.sync_copy(ones_v, deg_sh.at[dst4_v.at[j]],
                                    add=True)

        plsc.subcore_barrier()
        pltpu.sync_copy(acc_sh.at[pl.ds(t0, _TROWS)],
                        acc_out.at[c, pl.ds(t0, _TROWS)])
        if with_deg:
            pltpu.sync_copy(deg_sh.at[pl.ds(t0, _TROWS)],
                            deg_out.at[c, pl.ds(t0, _TROWS)])

    return pl.kernel(
        body,
        out_type=tuple(outs) if with_deg else outs[0],
        mesh=_sc_mesh, scratch_types=scratch, compiler_params=_sc_params)


def _make_dinv_expand():
    npt = _NPAD // _NTILE
    scratch = [
        pltpu.VMEM((npt,), jnp.float32),
        pltpu.VMEM((npt,), jnp.float32),
        pltpu.VMEM((npt * 16,), jnp.float32),
    ]

    def body(degp, out, d0_v, d1_v, out_v):
        c = lax.axis_index("c")
        s = lax.axis_index("s")
        base = (c * _NS + s) * npt
        pltpu.sync_copy(degp.at[0, pl.ds(base, npt)], d0_v)
        pltpu.sync_copy(degp.at[1, pl.ds(base, npt)], d1_v)

        dnums = lax.GatherDimensionNumbers(
            offset_dims=(), collapsed_slice_dims=(0,), start_index_map=(0,))

        def vr(i, carry):
            v = d0_v[pl.ds(i * 16, 16)] + d1_v[pl.ds(i * 16, 16)]
            dinv = 1.0 / jnp.maximum(v, 1.0)
            for lane in range(16):
                idx = jnp.full((16, 1), lane, jnp.int32)
                out_v[pl.ds((i * 16 + lane) * 16, 16)] = lax.gather(
                    dinv, idx, dnums, slice_sizes=(1,),
                    mode=lax.GatherScatterMode.PROMISE_IN_BOUNDS)
            return carry
        lax.fori_loop(0, npt // 16, vr, 0)
        pltpu.sync_copy(out_v, out.at[pl.ds(base * 16, npt * 16)])

    return pl.kernel(
        body, out_type=jax.ShapeDtypeStruct((_NPAD * 16,), jnp.float32),
        mesh=_sc_mesh, scratch_types=scratch, compiler_params=_sc_params)


_edge_pass_deg = _make_edge_pass(True)
_edge_pass = _make_edge_pass(False)
_dinv_expand = _make_dinv_expand()

_PROWS = _NPAD * _D // 128
_XROWS = _N * _D // 128
_PBLK = 256
_PGRID = _PROWS // _PBLK


def _tc1_body(p_ref, dinv_ref, x_ref, k1, b1, k2, b2, g_ref, r_ref):
    mean = (p_ref[0] + p_ref[1]) * dinv_ref[...]
    a = jnp.concatenate([mean, x_ref[...]], axis=1)
    h = jnp.dot(a, k1[...], preferred_element_type=jnp.float32) + b1[...]
    h = jnp.maximum(h, 0.0)
    gr = jnp.dot(h, k2[...], preferred_element_type=jnp.float32)
    g_ref[...] = gr[:, :128]
    r_ref[...] = gr[:, 128:] + b2[...]


def _tc2_body(q_ref, dinv_ref, r_ref, o_ref):
    o_ref[...] = (q_ref[0] + q_ref[1]) * dinv_ref[...] + r_ref[...]


_prow_spec = pl.BlockSpec((_PBLK, 128), lambda i: (i, 0))
_ppart_spec = pl.BlockSpec((_NC, _PBLK, 128), lambda i: (0, i, 0))


def _w_spec(r, c):
    return pl.BlockSpec((r, c), lambda i: (0, 0))


_tc1 = pl.pallas_call(
    _tc1_body,
    grid=(_PGRID,),
    in_specs=[
        _ppart_spec, _prow_spec, _prow_spec,
        _w_spec(256, 256), _w_spec(1, 256),
        _w_spec(256, 256), _w_spec(1, 128),
    ],
    out_specs=[_prow_spec, _prow_spec],
    out_shape=[jax.ShapeDtypeStruct((_PROWS, 128), jnp.float32)] * 2,
)

_tc2 = pl.pallas_call(
    _tc2_body,
    grid=(_PGRID,),
    in_specs=[_ppart_spec, _prow_spec, _prow_spec],
    out_specs=pl.BlockSpec((_PBLK, 128), lambda i: (i, 0)),
    out_shape=jax.ShapeDtypeStruct((_XROWS, 128), jnp.float32),
)


def kernel(x, edge_index, W1_l, b1_l, W1_r, W2_l, b2_l, W2_r):
    edges = edge_index.reshape(2, _EROWS, _LANE)
    x1 = x.reshape(-1)
    eye8 = jnp.eye(8, dtype=jnp.float32)
    k1 = jnp.concatenate(
        [jnp.kron(eye8, W1_l), jnp.kron(eye8, W1_r)], axis=0)
    k2 = jnp.concatenate(
        [jnp.kron(eye8, W2_l), jnp.kron(eye8, W2_r)], axis=1)
    b1t = jnp.tile(b1_l, 8).reshape(1, -1)
    b2t = jnp.tile(b2_l, 8).reshape(1, -1)

    p1, degp = _edge_pass_deg(edges, x1.reshape(_N, _D))
    dinv = _dinv_expand(degp).reshape(_PROWS, 128)
    g, r0 = _tc1(p1.reshape(_NC, _PROWS, 128), dinv,
                 x1.reshape(_XROWS, 128), k1, b1t, k2, b2t)
    p2 = _edge_pass(edges, g.reshape(_NPAD, _D))
    out = _tc2(p2.reshape(_NC, _PROWS, 128), dinv, r0)
    return out.reshape(-1).reshape(_N, _D)


# speedup vs baseline: 30.8293x; 1.1032x over previous
import functools

import jax
import jax.numpy as jnp
from jax import lax
from jax.experimental import pallas as pl
from jax.experimental.pallas import tpu as pltpu
from jax.experimental.pallas import tpu_sc as plsc

_N = 100000
_NPAD = 100352
_D = 16
_LANE = 128
_GRP = 8
_NC, _NS = 2, 16
_NTILE = _NC * _NS
_EROWS = 1600000 // _LANE
_NFULL = _EROWS // _GRP
_TAIL_ROW = _NFULL * _GRP
_TAIL = _EROWS - _TAIL_ROW
_TROWS = _NPAD // _NS
_ZROWS = 98

_sc_mesh = plsc.VectorSubcoreMesh(core_axis_name="c", subcore_axis_name="s")
_sc_params = pltpu.CompilerParams(use_tc_tiling_on_sc=False)


def _make_edge_pass(with_deg):
    outs = [jax.ShapeDtypeStruct((_NC, _NPAD, _D), jnp.float32)]
    scratch = [
        pltpu.VMEM_SHARED((_NPAD, _D), jnp.float32),
        pltpu.VMEM((_GRP, _LANE), jnp.int32),
        pltpu.VMEM((_GRP, _LANE), jnp.int32),
        pltpu.VMEM((_TAIL, _LANE), jnp.int32),
        pltpu.VMEM((_TAIL, _LANE), jnp.int32),
        pltpu.VMEM((_GRP, _LANE, _D), jnp.float32),
        pltpu.VMEM((_ZROWS, _D), jnp.float32),
        pltpu.SemaphoreType.DMA,
        pltpu.SemaphoreType.DMA,
        pltpu.SemaphoreType.DMA,
    ]
    if with_deg:
        outs.append(jax.ShapeDtypeStruct((_NC, _NPAD), jnp.float32))
        scratch += [
            pltpu.VMEM_SHARED((_NPAD,), jnp.float32),
            pltpu.VMEM((_TROWS // 8,), jnp.float32),
            pltpu.VMEM((_LANE,), jnp.float32),
        ]

    def body(edges, table, *rest):
        if with_deg:
            (acc_out, deg_out, acc_sh, src_v, dst_v, src4_v, dst4_v, rows_v,
             zacc_v, sem_a, sem_b, sem_c, deg_sh, zdeg_v, ones_v) = rest
        else:
            (acc_out, acc_sh, src_v, dst_v, src4_v, dst4_v, rows_v,
             zacc_v, sem_a, sem_b, sem_c) = rest
        c = lax.axis_index("c")
        s = lax.axis_index("s")
        w = c * _NS + s
        t0 = s * _TROWS

        def zrow(i, carry):
            zacc_v[i, :] = jnp.zeros((_D,), jnp.float32)
            return carry
        lax.fori_loop(0, _ZROWS, zrow, 0)

        def zcopy(g, carry):
            pltpu.async_copy(
                zacc_v, acc_sh.at[pl.ds(t0 + g * _ZROWS, _ZROWS)], sem_a)
            return carry
        lax.fori_loop(0, _TROWS // _ZROWS, zcopy, 0)

        def zdrain(g, carry):
            pltpu.make_async_copy(
                zacc_v, acc_sh.at[pl.ds(t0 + g * _ZROWS, _ZROWS)],
                sem_a).wait()
            return carry
        lax.fori_loop(0, _TROWS // _ZROWS, zdrain, 0)

        if with_deg:
            def zdrow(i, carry):
                zdeg_v[pl.ds(i * 16, 16)] = jnp.zeros((16,), jnp.float32)
                return carry
            lax.fori_loop(0, _TROWS // 8 // 16, zdrow, 0)

            def zdcopy(g, carry):
                pltpu.async_copy(
                    zdeg_v,
                    deg_sh.at[pl.ds(t0 + g * (_TROWS // 8), _TROWS // 8)],
                    sem_b)
                return carry
            lax.fori_loop(0, 8, zdcopy, 0)

            def zddrain(g, carry):
                pltpu.make_async_copy(
                    zdeg_v,
                    deg_sh.at[pl.ds(t0 + g * (_TROWS // 8), _TROWS // 8)],
                    sem_b).wait()
                return carry
            lax.fori_loop(0, 8, zddrain, 0)

            def orow(i, carry):
                ones_v[pl.ds(i * 16, 16)] = jnp.ones((16,), jnp.float32)
                return carry
            lax.fori_loop(0, _LANE // 16, orow, 0)

        plsc.subcore_barrier()

        ngroups = jnp.where(w < _NFULL % _NTILE, _NFULL // _NTILE + 1,
                            _NFULL // _NTILE)

        half = _GRP // 2

        def group(g, carry):
            r0 = (g * _NTILE + w) * _GRP
            ia = pltpu.async_copy(edges.at[0, pl.ds(r0, _GRP)], src_v, sem_a)
            ib = pltpu.async_copy(edges.at[1, pl.ds(r0, _GRP)], dst_v, sem_b)
            ia.wait()
            ib.wait()
            ga = [
                pltpu.async_copy(table.at[src_v.at[j]], rows_v.at[j], sem_a)
                for j in range(half)
            ]
            gb = [
                pltpu.async_copy(table.at[src_v.at[j]], rows_v.at[j], sem_b)
                for j in range(half, _GRP)
            ]
            sc = []
            for d in ga:
                d.wait()
            for j in range(half):
                sc.append(pltpu.async_copy(
                    rows_v.at[j], acc_sh.at[dst_v.at[j]], sem_c, add=True))
                if with_deg:
                    sc.append(pltpu.async_copy(
                        ones_v, deg_sh.at[dst_v.at[j]], sem_c, add=True))
            for d in gb:
                d.wait()
            for j in range(half, _GRP):
                sc.append(pltpu.async_copy(
                    rows_v.at[j], acc_sh.at[dst_v.at[j]], sem_c, add=True))
                if with_deg:
                    sc.append(pltpu.async_copy(
                        ones_v, deg_sh.at[dst_v.at[j]], sem_c, add=True))
            for d in sc:
                d.wait()
            return carry
        lax.fori_loop(0, ngroups, group, 0)

        @pl.when(w == _NTILE - 1)
        def _tail():
            pltpu.sync_copy(edges.at[0, pl.ds(_TAIL_ROW, _TAIL)], src4_v)
            pltpu.sync_copy(edges.at[1, pl.ds(_TAIL_ROW, _TAIL)], dst4_v)
            descs = [
                pltpu.async_copy(table.at[src4_v.at[j]], rows_v.at[j], sem_a)
                for j in range(_TAIL)
            ]
            for d in descs:
                d.wait()
            for j in range(_TAIL):
                pltpu.sync_copy(rows_v.at[j], acc_sh.at[dst4_v.at[j]],
                                add=True)
            if with_deg:
                for j in range(_TAIL):
                    pltpu.sync_copy(ones_v, deg_sh.at[dst4_v.at[j]],
                                    add=True)

        plsc.subcore_barrier()
        pltpu.sync_copy(acc_sh.at[pl.ds(t0, _TROWS)],
                        acc_out.at[c, pl.ds(t0, _TROWS)])
        if with_deg:
            pltpu.sync_copy(deg_sh.at[pl.ds(t0, _TROWS)],
                            deg_out.at[c, pl.ds(t0, _TROWS)])

    return pl.kernel(
        body,
        out_type=tuple(outs) if with_deg else outs[0],
        mesh=_sc_mesh, scratch_types=scratch, compiler_params=_sc_params)


def _make_dinv_expand():
    npt = _NPAD // _NTILE
    scratch = [
        pltpu.VMEM((npt,), jnp.float32),
        pltpu.VMEM((npt,), jnp.float32),
        pltpu.VMEM((npt * 16,), jnp.float32),
    ]

    def body(degp, out, d0_v, d1_v, out_v):
        c = lax.axis_index("c")
        s = lax.axis_index("s")
        base = (c * _NS + s) * npt
        pltpu.sync_copy(degp.at[0, pl.ds(base, npt)], d0_v)
        pltpu.sync_copy(degp.at[1, pl.ds(base, npt)], d1_v)

        dnums = lax.GatherDimensionNumbers(
            offset_dims=(), collapsed_slice_dims=(0,), start_index_map=(0,))

        def vr(i, carry):
            v = d0_v[pl.ds(i * 16, 16)] + d1_v[pl.ds(i * 16, 16)]
            dinv = 1.0 / jnp.maximum(v, 1.0)
            for lane in range(16):
                idx = jnp.full((16, 1), lane, jnp.int32)
                out_v[pl.ds((i * 16 + lane) * 16, 16)] = lax.gather(
                    dinv, idx, dnums, slice_sizes=(1,),
                    mode=lax.GatherScatterMode.PROMISE_IN_BOUNDS)
            return carry
        lax.fori_loop(0, npt // 16, vr, 0)
        pltpu.sync_copy(out_v, out.at[pl.ds(base * 16, npt * 16)])

    return pl.kernel(
        body, out_type=jax.ShapeDtypeStruct((_NPAD * 16,), jnp.float32),
        mesh=_sc_mesh, scratch_types=scratch, compiler_params=_sc_params)


_PROWS = _NPAD * _D // 128
_XROWS = _N * _D // 128
_PBLK = 256
_PGRID = _PROWS // _PBLK

_edge_pass_deg = _make_edge_pass(True)
_edge_pass = _make_edge_pass(False)
_dinv_expand = _make_dinv_expand()


def _tc1_body(p_ref, dinv_ref, x_ref, k1, b1, k2, b2, g_ref, r_ref):
    mean = (p_ref[0] + p_ref[1]) * dinv_ref[...]
    a = jnp.concatenate([mean, x_ref[...]], axis=1)
    h = jnp.dot(a, k1[...], preferred_element_type=jnp.float32) + b1[...]
    h = jnp.maximum(h, 0.0)
    gr = jnp.dot(h, k2[...], preferred_element_type=jnp.float32)
    g_ref[...] = gr[:, :128]
    r_ref[...] = gr[:, 128:] + b2[...]


def _tc2_body(q_ref, dinv_ref, r_ref, o_ref):
    o_ref[...] = (q_ref[0] + q_ref[1]) * dinv_ref[...] + r_ref[...]


_prow_spec = pl.BlockSpec((_PBLK, 128), lambda i: (i, 0))
_ppart_spec = pl.BlockSpec((_NC, _PBLK, 128), lambda i: (0, i, 0))


def _w_spec(r, c):
    return pl.BlockSpec((r, c), lambda i: (0, 0))


_tc1 = pl.pallas_call(
    _tc1_body,
    grid=(_PGRID,),
    in_specs=[
        _ppart_spec, _prow_spec, _prow_spec,
        _w_spec(256, 256), _w_spec(1, 256),
        _w_spec(256, 256), _w_spec(1, 128),
    ],
    out_specs=[_prow_spec, _prow_spec],
    out_shape=[jax.ShapeDtypeStruct((_PROWS, 128), jnp.float32)] * 2,
)

_tc2 = pl.pallas_call(
    _tc2_body,
    grid=(_PGRID,),
    in_specs=[_ppart_spec, _prow_spec, _prow_spec],
    out_specs=pl.BlockSpec((_PBLK, 128), lambda i: (i, 0)),
    out_shape=jax.ShapeDtypeStruct((_XROWS, 128), jnp.float32),
)


def kernel(x, edge_index, W1_l, b1_l, W1_r, W2_l, b2_l, W2_r):
    edges = edge_index.reshape(2, _EROWS, _LANE)
    x1 = x.reshape(-1)
    eye8 = jnp.eye(8, dtype=jnp.float32)
    k1 = jnp.concatenate(
        [jnp.kron(eye8, W1_l), jnp.kron(eye8, W1_r)], axis=0)
    k2 = jnp.concatenate(
        [jnp.kron(eye8, W2_l), jnp.kron(eye8, W2_r)], axis=1)
    b1t = jnp.tile(b1_l, 8).reshape(1, -1)
    b2t = jnp.tile(b2_l, 8).reshape(1, -1)

    p1, degp = _edge_pass_deg(edges, x1.reshape(_N, _D))
    dinv = _dinv_expand(degp).reshape(_PROWS, 128)
    g, r0 = _tc1(p1.reshape(_NC, _PROWS, 128), dinv,
                 x1.reshape(_XROWS, 128), k1, b1t, k2, b2t)
    p2 = _edge_pass(edges, g.reshape(_NPAD, _D))
    out = _tc2(p2.reshape(_NC, _PROWS, 128), dinv, r0)
    return out.reshape(-1).reshape(_N, _D)
